# baseline (speedup 1.0000x reference)
import functools

import jax
import jax.numpy as jnp
from jax import lax
from jax.experimental import pallas as pl
from jax.experimental.pallas import tpu as pltpu
from jax.experimental.pallas import tpu_sc as plsc


@functools.lru_cache(maxsize=None)
def _make_gather(n_b, n_l, n_v, d):
    info = plsc.get_sparse_core_info()
    NC, NS, L = info.num_cores, info.num_subcores, info.num_lanes
    NW = NC * NS
    BW = n_b // NW
    assert n_b % NW == 0 and BW % L == 0 and d % L == 0 and n_l % 2 == 0
    KG = BW // L
    mesh = plsc.VectorSubcoreMesh(core_axis_name="c", subcore_axis_name="s")

    @functools.partial(
        pl.kernel,
        mesh=mesh,
        compiler_params=pltpu.CompilerParams(
            use_tc_tiling_on_sc=True, needs_layout_passes=False
        ),
        out_type=jax.ShapeDtypeStruct((n_l, d, n_b), jnp.float32),
        scratch_types=(
            [
                pltpu.VMEM((n_l, BW), jnp.int32),
                pltpu.VMEM((2, BW), jnp.int32),
                pltpu.VMEM((2, BW), jnp.int32),
                pltpu.VMEM((BW, 128), jnp.float32),
                pltpu.VMEM((BW, 128), jnp.float32),
                pltpu.VMEM((1, d, BW), jnp.float32),
                pltpu.VMEM((1, d, BW), jnp.float32),
                pltpu.SemaphoreType.DMA,
            ]
            + [pltpu.SemaphoreType.DMA] * 2
            + [pltpu.SemaphoreType.DMA] * 2
        ),
    )
    def gather_kernel(
        idxt_hbm, emb2_hbm, out_hbm,
        idxt_v, hl_v, po_v, rows0, rows1, tb0, tb1,
        si, sg0, sg1, ss0, ss1,
    ):
        rows = (rows0, rows1)
        tbs = (tb0, tb1)
        sg = (sg0, sg1)
        ss = (ss0, ss1)
        wid = lax.axis_index("s") * NC + lax.axis_index("c")
        cb = wid * BW
        iota = lax.iota(jnp.int32, L)

        pltpu.sync_copy(idxt_hbm.at[:, pl.ds(cb, BW)], idxt_v)

        def prep(g, s):
            for k in range(KG):
                sl = pl.ds(k * L, L)
                v = jnp.maximum(idxt_v[g, sl], 0)
                hl_v[s, sl] = lax.shift_right_logical(v, 1)
                po_v[s, sl] = lax.shift_left(jnp.bitwise_and(v, 1), 6)

        def start_gather(g, s):
            pltpu.async_copy(emb2_hbm.at[hl_v.at[s]], rows[s], sg[s])

        def wait_gather(g, s):
            pltpu.make_async_copy(emb2_hbm.at[hl_v.at[s]], rows[s], sg[s]).wait()

        def select_transpose(s):
            for k in range(KG):
                jvec = iota + (k * L)
                po_k = po_v[s, pl.ds(k * L, L)]
                for dd in range(d):
                    vals = plsc.load_gather(rows[s], [jvec, po_k + dd])
                    tbs[s][0, dd, pl.ds(k * L, L)] = vals

        def start_store(g, s):
            pltpu.async_copy(
                tbs[s], out_hbm.at[pl.ds(g, 1), :, pl.ds(cb, BW)], ss[s]
            )

        def wait_store(g, s):
            pltpu.make_async_copy(
                tbs[s], out_hbm.at[pl.ds(g, 1), :, pl.ds(cb, BW)], ss[s]
            ).wait()

        for j in range(2):
            prep(j, j)
            start_gather(j, j)

        n_groups = n_l // 2

        def group_body(ng, carry):
            for j in range(2):
                g = 2 * ng + j
                wait_gather(g, j)

                @pl.when(ng >= 1)
                def _():
                    wait_store(g - 2, j)

                select_transpose(j)
                start_store(g, j)

                @pl.when(ng < n_groups - 1)
                def _():
                    prep(g + 2, j)
                    start_gather(g + 2, j)

            return carry

        lax.fori_loop(0, n_groups, group_body, 0)

        for j in range(2):
            wait_store(n_l - 2 + j, j)

    return gather_kernel


def kernel(idx, emb):
    b, l = idx.shape
    n_v, d = emb.shape
    idxt = idx.T.astype(jnp.int32)
    emb2 = emb.reshape(n_v * d // 128, 128)
    out_t = _make_gather(b, l, n_v, d)(idxt, emb2)
    return out_t.transpose(2, 0, 1)


# speedup vs baseline: 1.5776x; 1.5776x over previous
import functools

import jax
import jax.numpy as jnp
from jax import lax
from jax.experimental import pallas as pl
from jax.experimental.pallas import tpu as pltpu
from jax.experimental.pallas import tpu_sc as plsc

NBUF = 8


@functools.lru_cache(maxsize=None)
def _make_gather(n_b, n_l, d):
    info = plsc.get_sparse_core_info()
    NC, NS, L = info.num_cores, info.num_subcores, info.num_lanes
    NW = NC * NS
    assert n_b % (NW * NBUF) == 0 and (n_b * n_l) % (NW * L) == 0
    b_per_w = n_b // NW
    per_w = b_per_w * n_l
    n_groups = b_per_w // NBUF
    mesh = plsc.VectorSubcoreMesh(core_axis_name="c", subcore_axis_name="s")

    @functools.partial(
        pl.kernel,
        mesh=mesh,
        compiler_params=pltpu.CompilerParams(use_tc_tiling_on_sc=False),
        out_type=jax.ShapeDtypeStruct((n_b, n_l, d), jnp.float32),
        scratch_types=(
            [pltpu.VMEM((per_w,), jnp.int32)]
            + [pltpu.VMEM((1, n_l, d), jnp.float32)] * NBUF
            + [pltpu.SemaphoreType.DMA] * (2 * NBUF)
        ),
    )
    def gather_kernel(idx_hbm, emb_hbm, out_hbm, idx_v, *bufs):
        rows = bufs[:NBUF]
        sg = bufs[NBUF : 2 * NBUF]
        ss = bufs[2 * NBUF : 3 * NBUF]
        wid = lax.axis_index("s") * NC + lax.axis_index("c")
        base = wid * per_w
        bbase = wid * b_per_w

        pltpu.sync_copy(idx_hbm.at[pl.ds(base, per_w)], idx_v)

        def clamp_all():
            def body(j, c):
                sl = pl.ds(j * L, L)
                idx_v[sl] = jnp.maximum(idx_v[sl], 0)
                return c

            lax.fori_loop(0, per_w // L, body, 0, unroll=8)

        def start_gather(g, b):
            pltpu.async_copy(
                emb_hbm.at[idx_v.at[pl.ds(g * n_l, n_l)]], rows[b].at[0], sg[b]
            )

        def wait_gather(g, b):
            pltpu.make_async_copy(
                emb_hbm.at[idx_v.at[pl.ds(g * n_l, n_l)]], rows[b].at[0], sg[b]
            ).wait()

        def start_store(g, b):
            pltpu.async_copy(rows[b], out_hbm.at[pl.ds(bbase + g, 1)], ss[b])

        def wait_store(g, b):
            pltpu.make_async_copy(
                rows[b], out_hbm.at[pl.ds(bbase + g, 1)], ss[b]
            ).wait()

        clamp_all()

        for j in range(NBUF):
            start_gather(j, j)
            if j >= 1:
                wait_gather(j - 1, j - 1)
                start_store(j - 1, j - 1)

        def group_body(ng, carry):
            for j in range(NBUF):
                g = ng * NBUF + j
                wait_store(g - NBUF, j)
                start_gather(g, j)
                wait_gather(g - 1, (j - 1) % NBUF)
                start_store(g - 1, (j - 1) % NBUF)
            return carry

        lax.fori_loop(1, n_groups, group_body, 0)

        last = b_per_w - 1
        wait_gather(last, NBUF - 1)
        start_store(last, NBUF - 1)
        for j in range(NBUF):
            wait_store(b_per_w - NBUF + j, j)

    return gather_kernel


def kernel(idx, emb):
    b, l = idx.shape
    d = emb.shape[1]
    flat = idx.reshape(b * l).astype(jnp.int32)
    return _make_gather(b, l, d)(flat, emb)
